# baseline (speedup 1.0000x reference)
import functools

import jax
import jax.numpy as jnp
from jax import lax
from jax.experimental import pallas as pl
from jax.experimental.pallas import tpu as pltpu
from jax.experimental.pallas import tpu_sc as plsc

N = 8192
NW = 32
QPW = N // NW
QBLK = 4
LANES = 16
NCH = 3
NCHUNK = N // LANES


def _sc_body(at_hbm, bt_hbm, rowsum_hbm, colacc_hbm,
             a_v, b_v, colacc_v, out_v):
    wid = lax.axis_index("s") * 2 + lax.axis_index("c")
    base = wid * QPW

    for c in range(NCH):
        pltpu.sync_copy(at_hbm.at[pl.ds(c * N + base, QPW)],
                        a_v.at[pl.ds(c * QPW, QPW)])
    pltpu.sync_copy(bt_hbm, b_v)

    inf16 = jnp.full((LANES,), jnp.inf, dtype=jnp.float32)

    def init_body(i, carry):
        colacc_v[pl.ds(i * LANES, LANES)] = inf16
        return carry

    lax.fori_loop(0, NCH * NCHUNK, init_body, 0, unroll=8)

    lane0 = lax.iota(jnp.int32, LANES) == 0
    zeros16 = jnp.zeros((LANES,), dtype=jnp.float32)
    total = zeros16
    for c in range(NCH):
        def qgroup(g, rowsum, c=c):
            avec = a_v[pl.ds(c * QPW + (g // 4) * LANES, LANES)]
            lane0q = (g % 4) * QBLK
            gdn = lax.GatherDimensionNumbers(
                offset_dims=(), collapsed_slice_dims=(0,),
                start_index_map=(0,))
            ab = [lax.gather(avec,
                             jnp.full((LANES, 1), lane0q + q, jnp.int32),
                             gdn, (1,),
                             mode=lax.GatherScatterMode.PROMISE_IN_BOUNDS)
                  for q in range(QBLK)]

            def sweep(j, accs, c=c, ab=ab):
                bv = b_v[pl.ds(c * N + j * LANES, LANES)]
                ds = [jnp.abs(bv - ab[q]) for q in range(QBLK)]
                cm = jnp.minimum(jnp.minimum(ds[0], ds[1]),
                                 jnp.minimum(ds[2], ds[3]))
                off = c * N + j * LANES
                colacc_v[pl.ds(off, LANES)] = jnp.minimum(
                    colacc_v[pl.ds(off, LANES)], cm)
                return tuple(jnp.minimum(accs[q], ds[q]) for q in range(QBLK))

            accs = lax.fori_loop(0, NCHUNK, sweep,
                                 (inf16,) * QBLK, unroll=2)
            for q in range(QBLK):
                smin, _ = plsc.sort_key_val(accs[q], accs[q])
                rowsum = rowsum + jnp.where(lane0, smin, 0.0)
            return rowsum

        total = lax.fori_loop(0, QPW // QBLK, qgroup, total)

    out_v[...] = total
    pltpu.sync_copy(out_v, rowsum_hbm.at[wid])
    pltpu.sync_copy(colacc_v, colacc_hbm.at[wid])


def _sc_partials(at, bt):
    mesh = plsc.VectorSubcoreMesh(core_axis_name="c", subcore_axis_name="s")
    f = functools.partial(
        pl.kernel,
        out_type=(
            jax.ShapeDtypeStruct((NW, LANES), jnp.float32),
            jax.ShapeDtypeStruct((NW, NCH * N), jnp.float32),
        ),
        mesh=mesh,
        scratch_types=[
            pltpu.VMEM((NCH * QPW,), jnp.float32),
            pltpu.VMEM((NCH * N,), jnp.float32),
            pltpu.VMEM((NCH * N,), jnp.float32),
            pltpu.VMEM((LANES,), jnp.float32),
        ],
        compiler_params=pltpu.CompilerParams(needs_layout_passes=False),
    )(_sc_body)
    return f(at, bt)


COMB_CHUNK = 128
N_COMB = (NCH * N) // COMB_CHUNK


def _combine_body(rowsum_ref, colacc_ref, out_ref, acc_ref):
    k = pl.program_id(0)

    @pl.when(k == 0)
    def _init():
        acc_ref[0] = 0.0

    colmin = jnp.min(colacc_ref[...], axis=0)
    acc_ref[0] += jnp.sum(colmin)

    @pl.when(k == N_COMB - 1)
    def _finish():
        out_ref[0, 0] = (acc_ref[0] + jnp.sum(rowsum_ref[...])) / N


def _combine(rowsum, colacc):
    return pl.pallas_call(
        _combine_body,
        grid=(N_COMB,),
        in_specs=[
            pl.BlockSpec((NW, LANES), lambda k: (0, 0)),
            pl.BlockSpec((NW, COMB_CHUNK), lambda k: (0, k)),
        ],
        out_specs=pl.BlockSpec(
            (1, 1), lambda k: (0, 0), memory_space=pltpu.SMEM),
        out_shape=jax.ShapeDtypeStruct((1, 1), jnp.float32),
        scratch_shapes=[pltpu.SMEM((1,), jnp.float32)],
        compiler_params=pltpu.CompilerParams(
            dimension_semantics=("arbitrary",)),
    )(rowsum, colacc)


@jax.jit
def kernel(pred, target):
    at = pred.T.reshape(NCH * N)
    bt = target.T.reshape(NCH * N)
    rowsum, colacc = _sc_partials(at, bt)
    out = _combine(rowsum, colacc)
    return out[0, 0]


# speedup vs baseline: 1.9829x; 1.9829x over previous
import functools

import jax
import jax.numpy as jnp
from jax import lax
from jax.experimental import pallas as pl
from jax.experimental.pallas import tpu as pltpu
from jax.experimental.pallas import tpu_sc as plsc

N = 8192
NW = 32
QPW = N // NW
QBLK = 16
LANES = 16
NCH = 3
NCHUNK = N // LANES


def _sc_body(at_hbm, bt_hbm, rowsum_hbm, colacc_hbm,
             a_v, b_v, colacc_v, out_v):
    wid = lax.axis_index("s") * 2 + lax.axis_index("c")
    base = wid * QPW

    for c in range(NCH):
        pltpu.sync_copy(at_hbm.at[pl.ds(c * N + base, QPW)],
                        a_v.at[pl.ds(c * QPW, QPW)])
    pltpu.sync_copy(bt_hbm, b_v)

    inf16 = jnp.full((LANES,), jnp.inf, dtype=jnp.float32)

    def init_body(i, carry):
        colacc_v[pl.ds(i * LANES, LANES)] = inf16
        return carry

    lax.fori_loop(0, NCH * NCHUNK, init_body, 0, unroll=8)

    lane0 = lax.iota(jnp.int32, LANES) == 0
    zeros16 = jnp.zeros((LANES,), dtype=jnp.float32)
    total = zeros16
    for c in range(NCH):
        def qgroup(g, rowsum, c=c):
            avec = a_v[pl.ds(c * QPW + g * LANES, LANES)]
            gdn = lax.GatherDimensionNumbers(
                offset_dims=(), collapsed_slice_dims=(0,),
                start_index_map=(0,))
            ab = [lax.gather(avec,
                             jnp.full((LANES, 1), q, jnp.int32),
                             gdn, (1,),
                             mode=lax.GatherScatterMode.PROMISE_IN_BOUNDS)
                  for q in range(QBLK)]

            def sweep(j, accs, c=c, ab=ab):
                bv = b_v[pl.ds(c * N + j * LANES, LANES)]
                ds = [jnp.abs(bv - ab[q]) for q in range(QBLK)]
                t = ds
                while len(t) > 1:
                    t = [jnp.minimum(t[2 * i], t[2 * i + 1])
                         for i in range(len(t) // 2)]
                off = c * N + j * LANES
                colacc_v[pl.ds(off, LANES)] = jnp.minimum(
                    colacc_v[pl.ds(off, LANES)], t[0])
                return tuple(jnp.minimum(accs[q], ds[q]) for q in range(QBLK))

            accs = lax.fori_loop(0, NCHUNK, sweep,
                                 (inf16,) * QBLK, unroll=1)
            for q in range(QBLK):
                smin, _ = plsc.sort_key_val(accs[q], accs[q])
                rowsum = rowsum + jnp.where(lane0, smin, 0.0)
            return rowsum

        total = lax.fori_loop(0, QPW // QBLK, qgroup, total)

    out_v[...] = total
    pltpu.sync_copy(out_v, rowsum_hbm.at[wid])
    pltpu.sync_copy(colacc_v, colacc_hbm.at[wid])


def _sc_partials(at, bt):
    mesh = plsc.VectorSubcoreMesh(core_axis_name="c", subcore_axis_name="s")
    f = functools.partial(
        pl.kernel,
        out_type=(
            jax.ShapeDtypeStruct((NW, LANES), jnp.float32),
            jax.ShapeDtypeStruct((NW, NCH * N), jnp.float32),
        ),
        mesh=mesh,
        scratch_types=[
            pltpu.VMEM((NCH * QPW,), jnp.float32),
            pltpu.VMEM((NCH * N,), jnp.float32),
            pltpu.VMEM((NCH * N,), jnp.float32),
            pltpu.VMEM((LANES,), jnp.float32),
        ],
        compiler_params=pltpu.CompilerParams(needs_layout_passes=False),
    )(_sc_body)
    return f(at, bt)


COMB_CHUNK = 128
N_COMB = (NCH * N) // COMB_CHUNK


def _combine_body(rowsum_ref, colacc_ref, out_ref, acc_ref):
    k = pl.program_id(0)

    @pl.when(k == 0)
    def _init():
        acc_ref[0] = 0.0

    colmin = jnp.min(colacc_ref[...], axis=0)
    acc_ref[0] += jnp.sum(colmin)

    @pl.when(k == N_COMB - 1)
    def _finish():
        out_ref[0, 0] = (acc_ref[0] + jnp.sum(rowsum_ref[...])) / N


def _combine(rowsum, colacc):
    return pl.pallas_call(
        _combine_body,
        grid=(N_COMB,),
        in_specs=[
            pl.BlockSpec((NW, LANES), lambda k: (0, 0)),
            pl.BlockSpec((NW, COMB_CHUNK), lambda k: (0, k)),
        ],
        out_specs=pl.BlockSpec(
            (1, 1), lambda k: (0, 0), memory_space=pltpu.SMEM),
        out_shape=jax.ShapeDtypeStruct((1, 1), jnp.float32),
        scratch_shapes=[pltpu.SMEM((1,), jnp.float32)],
        compiler_params=pltpu.CompilerParams(
            dimension_semantics=("arbitrary",)),
    )(rowsum, colacc)


@jax.jit
def kernel(pred, target):
    at = pred.T.reshape(NCH * N)
    bt = target.T.reshape(NCH * N)
    rowsum, colacc = _sc_partials(at, bt)
    out = _combine(rowsum, colacc)
    return out[0, 0]


# speedup vs baseline: 3.3321x; 1.6804x over previous
import functools

import jax
import jax.numpy as jnp
from jax import lax
from jax.experimental import pallas as pl
from jax.experimental.pallas import tpu as pltpu
from jax.experimental.pallas import tpu_sc as plsc

N = 8192
NCH = 3
LANES = 16

TC_ROWS = 5632
SC_ROWS = N - TC_ROWS

NW = 32
QPW = SC_ROWS // NW
QBLK = 16
NCHUNK = N // LANES

ROW_BLK = 128
COL_CHUNK = 128
N_ROW_BLKS = TC_ROWS // ROW_BLK
N_COL_CHUNKS = N // COL_CHUNK


def _tc_body(a_ref, b_ref, rowsum_ref, colacc_ref):
    c = pl.program_id(0)
    r = pl.program_id(1)

    @pl.when((c == 0) & (r == 0))
    def _init():
        rowsum_ref[0, 0] = 0.0
        colacc_ref[...] = jnp.full(
            (NCH * N_COL_CHUNKS, 8, COL_CHUNK), jnp.inf, dtype=jnp.float32)

    a = jnp.broadcast_to(a_ref[0], (ROW_BLK, COL_CHUNK))

    def chunk_step(k, rowacc):
        bc = b_ref[0, pl.ds(k, 1), :]
        d = jnp.abs(a - bc)
        rowacc = jnp.minimum(rowacc, d)
        colpart = jnp.min(
            d.reshape(ROW_BLK // 8, 8, COL_CHUNK), axis=0)
        row = c * N_COL_CHUNKS + k
        colacc_ref[pl.ds(row, 1), :, :] = jnp.minimum(
            colacc_ref[pl.ds(row, 1), :, :], colpart[None])
        return rowacc

    rowacc0 = jnp.full((ROW_BLK, COL_CHUNK), jnp.inf, dtype=jnp.float32)
    rowacc = jax.lax.fori_loop(0, N_COL_CHUNKS, chunk_step, rowacc0,
                               unroll=4)
    rowmin = jnp.min(rowacc, axis=1)
    rowsum_ref[0, 0] += jnp.sum(rowmin)


def _tc_partials(a3, b3):
    return pl.pallas_call(
        _tc_body,
        grid=(NCH, N_ROW_BLKS),
        in_specs=[
            pl.BlockSpec((1, ROW_BLK, 1), lambda c, r: (c, r, 0)),
            pl.BlockSpec((1, N_COL_CHUNKS, COL_CHUNK), lambda c, r: (c, 0, 0)),
        ],
        out_specs=[
            pl.BlockSpec((1, 1), lambda c, r: (0, 0),
                         memory_space=pltpu.SMEM),
            pl.BlockSpec((NCH * N_COL_CHUNKS, 8, COL_CHUNK),
                         lambda c, r: (0, 0, 0)),
        ],
        out_shape=[
            jax.ShapeDtypeStruct((1, 1), jnp.float32),
            jax.ShapeDtypeStruct((NCH * N_COL_CHUNKS, 8, COL_CHUNK),
                                 jnp.float32),
        ],
        compiler_params=pltpu.CompilerParams(
            dimension_semantics=("arbitrary", "arbitrary")),
    )(a3, b3)


def _sc_body(at_hbm, bt_hbm, rowsum_hbm, colacc_hbm,
             a_v, b_v, colacc_v, out_v):
    wid = lax.axis_index("s") * 2 + lax.axis_index("c")
    base = TC_ROWS + wid * QPW

    for c in range(NCH):
        pltpu.sync_copy(at_hbm.at[pl.ds(c * N + base, QPW)],
                        a_v.at[pl.ds(c * QPW, QPW)])
    pltpu.sync_copy(bt_hbm, b_v)

    inf16 = jnp.full((LANES,), jnp.inf, dtype=jnp.float32)

    def init_body(i, carry):
        colacc_v[pl.ds(i * LANES, LANES)] = inf16
        return carry

    lax.fori_loop(0, NCH * NCHUNK, init_body, 0, unroll=8)

    lane0 = lax.iota(jnp.int32, LANES) == 0
    total = jnp.zeros((LANES,), dtype=jnp.float32)
    for c in range(NCH):
        def qgroup(g, rowsum, c=c):
            avec = a_v[pl.ds(c * QPW + g * LANES, LANES)]
            gdn = lax.GatherDimensionNumbers(
                offset_dims=(), collapsed_slice_dims=(0,),
                start_index_map=(0,))
            ab = [lax.gather(avec,
                             jnp.full((LANES, 1), q, jnp.int32),
                             gdn, (1,),
                             mode=lax.GatherScatterMode.PROMISE_IN_BOUNDS)
                  for q in range(QBLK)]

            def sweep(j, accs, c=c, ab=ab):
                bv = b_v[pl.ds(c * N + j * LANES, LANES)]
                ds = [jnp.abs(bv - ab[q]) for q in range(QBLK)]
                t = ds
                while len(t) > 1:
                    t = [jnp.minimum(t[2 * i], t[2 * i + 1])
                         for i in range(len(t) // 2)]
                off = c * N + j * LANES
                colacc_v[pl.ds(off, LANES)] = jnp.minimum(
                    colacc_v[pl.ds(off, LANES)], t[0])
                return tuple(jnp.minimum(accs[q], ds[q]) for q in range(QBLK))

            accs = lax.fori_loop(0, NCHUNK, sweep, (inf16,) * QBLK)
            for q in range(QBLK):
                smin, _ = plsc.sort_key_val(accs[q], accs[q])
                rowsum = rowsum + jnp.where(lane0, smin, 0.0)
            return rowsum

        total = lax.fori_loop(0, QPW // QBLK, qgroup, total)

    out_v[...] = total
    pltpu.sync_copy(out_v, rowsum_hbm.at[wid])
    pltpu.sync_copy(colacc_v, colacc_hbm.at[wid])


def _sc_partials(at, bt):
    mesh = plsc.VectorSubcoreMesh(core_axis_name="c", subcore_axis_name="s")
    f = functools.partial(
        pl.kernel,
        out_type=(
            jax.ShapeDtypeStruct((NW, LANES), jnp.float32),
            jax.ShapeDtypeStruct((NW, NCH * N), jnp.float32),
        ),
        mesh=mesh,
        scratch_types=[
            pltpu.VMEM((NCH * QPW,), jnp.float32),
            pltpu.VMEM((NCH * N,), jnp.float32),
            pltpu.VMEM((NCH * N,), jnp.float32),
            pltpu.VMEM((LANES,), jnp.float32),
        ],
        compiler_params=pltpu.CompilerParams(needs_layout_passes=False),
    )(_sc_body)
    return f(at, bt)


N_COMB = (NCH * N) // COL_CHUNK


def _combine_body(rowsum_sc_ref, rowsum_tc_ref, colacc_tc_ref,
                  colacc_sc_ref, out_ref, acc_ref):
    k = pl.program_id(0)

    @pl.when(k == 0)
    def _init():
        acc_ref[0] = 0.0

    tcm = jnp.min(colacc_tc_ref[0], axis=0, keepdims=True)
    scm = jnp.min(colacc_sc_ref[...], axis=0, keepdims=True)
    acc_ref[0] += jnp.sum(jnp.minimum(tcm, scm))

    @pl.when(k == N_COMB - 1)
    def _finish():
        out_ref[0, 0] = (acc_ref[0] + rowsum_tc_ref[0, 0]
                         + jnp.sum(rowsum_sc_ref[...])) / N


def _combine(rowsum_sc, rowsum_tc, colacc_tc, colacc_sc):
    return pl.pallas_call(
        _combine_body,
        grid=(N_COMB,),
        in_specs=[
            pl.BlockSpec((NW, LANES), lambda k: (0, 0)),
            pl.BlockSpec((1, 1), lambda k: (0, 0),
                         memory_space=pltpu.SMEM),
            pl.BlockSpec((1, 8, COL_CHUNK), lambda k: (k, 0, 0)),
            pl.BlockSpec((NW, COL_CHUNK), lambda k: (0, k)),
        ],
        out_specs=pl.BlockSpec(
            (1, 1), lambda k: (0, 0), memory_space=pltpu.SMEM),
        out_shape=jax.ShapeDtypeStruct((1, 1), jnp.float32),
        scratch_shapes=[pltpu.SMEM((1,), jnp.float32)],
        compiler_params=pltpu.CompilerParams(
            dimension_semantics=("arbitrary",)),
    )(rowsum_sc, rowsum_tc, colacc_tc, colacc_sc)


@jax.jit
def kernel(pred, target):
    at = pred.T
    bt = target.T
    a3 = at[:, :TC_ROWS, None]
    b3 = bt.reshape(NCH, N_COL_CHUNKS, COL_CHUNK)
    at_flat = at.reshape(NCH * N)
    bt_flat = bt.reshape(NCH * N)
    rowsum_sc, colacc_sc = _sc_partials(at_flat, bt_flat)
    rowsum_tc, colacc_tc = _tc_partials(a3, b3)
    out = _combine(rowsum_sc, rowsum_tc, colacc_tc, colacc_sc)
    return out[0, 0]
